# Initial kernel scaffold; baseline (speedup 1.0000x reference)
import jax
import jax.numpy as jnp
from jax.experimental import pallas as pl


def kernel(X, edge_index, Wx_l, Wx_r, bx, Wh_l, Wh_r, bh, w_c, b_gate):
    raise NotImplementedError("write your pallas kernel here")


# speedup vs baseline: 7.7867x; 7.7867x over previous
import functools

import jax
import jax.numpy as jnp
from jax import lax
from jax.experimental import pallas as pl
from jax.experimental.pallas import tpu as pltpu
from jax.experimental.pallas import tpu_sc as plsc

N = 10000
E = 320000
D = 128
NP = 10240
NC = 2
NS = 16
NW = NC * NS
EPW = E // NW
K = 80
NCHUNK = EPW // K
ROWS_PER_TILE = NP // NS


def _sc_body(x_hbm, src_hbm, dst_hbm, agg_out, cnt_out,
             agg_sh, cnt_sh, sidx, didx, rows, ones_v, zbuf, zcnt):
    cid = lax.axis_index("c")
    sid = lax.axis_index("s")
    wid = sid * NC + cid

    z16 = jnp.zeros((16,), jnp.float32)

    def _zb(i, _):
        for j in range(8):
            zbuf[i, pl.ds(j * 16, 16)] = z16
        return 0
    lax.fori_loop(0, 128, _zb, 0)

    def _zc(i, _):
        zcnt[pl.ds(i * 16, 16)] = z16
        return 0
    lax.fori_loop(0, ROWS_PER_TILE // 16, _zc, 0)

    def _on(i, _):
        ones_v[pl.ds(i * 16, 16)] = jnp.ones((16,), jnp.float32)
        return 0
    lax.fori_loop(0, K // 16, _on, 0)

    row0 = sid * ROWS_PER_TILE
    for t in range(ROWS_PER_TILE // 128):
        pltpu.sync_copy(zbuf, agg_sh.at[pl.ds(row0 + t * 128, 128)])
    pltpu.sync_copy(zcnt, cnt_sh.at[pl.ds(row0, ROWS_PER_TILE)])
    plsc.subcore_barrier()

    ebase = wid * EPW

    def _chunk(j, _):
        base = ebase + j * K
        pltpu.sync_copy(src_hbm.at[pl.ds(base, K)], sidx)
        pltpu.sync_copy(dst_hbm.at[pl.ds(base, K)], didx)
        pltpu.sync_copy(x_hbm.at[sidx], rows)
        pltpu.sync_copy(rows, agg_sh.at[didx], add=True)
        pltpu.sync_copy(ones_v, cnt_sh.at[didx], add=True)
        return 0
    lax.fori_loop(0, NCHUNK, _chunk, 0)

    plsc.subcore_barrier()

    pltpu.sync_copy(agg_sh.at[pl.ds(row0, ROWS_PER_TILE)],
                    agg_out.at[cid, pl.ds(row0, ROWS_PER_TILE)])
    pltpu.sync_copy(cnt_sh.at[pl.ds(row0, ROWS_PER_TILE)],
                    cnt_out.at[cid, pl.ds(row0, ROWS_PER_TILE)])


@functools.cache
def _sc_segment_sum():
    return pl.kernel(
        _sc_body,
        out_type=(jax.ShapeDtypeStruct((NC, NP, D), jnp.float32),
                  jax.ShapeDtypeStruct((NC, NP), jnp.float32)),
        mesh=plsc.VectorSubcoreMesh(core_axis_name="c", subcore_axis_name="s",
                                    num_cores=NC, num_subcores=NS),
        scratch_types=(
            pltpu.VMEM_SHARED((NP, D), jnp.float32),
            pltpu.VMEM_SHARED((NP,), jnp.float32),
            pltpu.VMEM((K,), jnp.int32),
            pltpu.VMEM((K,), jnp.int32),
            pltpu.VMEM((K, D), jnp.float32),
            pltpu.VMEM((K,), jnp.float32),
            pltpu.VMEM((128, D), jnp.float32),
            pltpu.VMEM((ROWS_PER_TILE,), jnp.float32),
        ),
    )


BLK = 2000


def _dense_body(x_ref, agg_ref, cnt_ref, wl_ref, wr_ref, bx_ref, bh_ref,
                wc_ref, bg_ref, h_ref, c_ref):
    X = x_ref[...]
    agg = agg_ref[0] + agg_ref[1]
    cnt = cnt_ref[0] + cnt_ref[1]
    mean = agg / jnp.maximum(cnt, 1.0)

    def gate(k):
        U = (jnp.dot(mean, wl_ref[k], preferred_element_type=jnp.float32)
             + jnp.dot(X, wr_ref[k], preferred_element_type=jnp.float32)
             + bx_ref[k])
        nrm = jnp.sqrt(jnp.sum(U * U, axis=1, keepdims=True))
        S = U / jnp.maximum(nrm, 1e-12)
        bhk = bh_ref[k]
        bnrm = jnp.sqrt(jnp.sum(bhk * bhk))
        return S + bhk / jnp.maximum(bnrm, 1e-12) + bg_ref[k]

    I = jax.nn.sigmoid(gate(0))
    T = jnp.tanh(gate(2))
    C = I * T
    O = jax.nn.sigmoid(gate(3) + wc_ref[2] * C)
    h_ref[...] = O * jnp.tanh(C)
    c_ref[...] = C


def kernel(X, edge_index, Wx_l, Wx_r, bx, Wh_l, Wh_r, bh, w_c, b_gate):
    del Wh_l, Wh_r
    edges = edge_index.astype(jnp.int32)
    agg2, cnt2 = _sc_segment_sum()(X, edges[0], edges[1])
    cnt2r = cnt2.reshape(NC, NP, 1)

    H, C = pl.pallas_call(
        _dense_body,
        grid=(N // BLK,),
        in_specs=[
            pl.BlockSpec((BLK, D), lambda i: (i, 0)),
            pl.BlockSpec((NC, BLK, D), lambda i: (0, i, 0)),
            pl.BlockSpec((NC, BLK, 1), lambda i: (0, i, 0)),
            pl.BlockSpec((4, D, D), lambda i: (0, 0, 0)),
            pl.BlockSpec((4, D, D), lambda i: (0, 0, 0)),
            pl.BlockSpec((4, D), lambda i: (0, 0)),
            pl.BlockSpec((4, D), lambda i: (0, 0)),
            pl.BlockSpec((3, D), lambda i: (0, 0)),
            pl.BlockSpec((4, D), lambda i: (0, 0)),
        ],
        out_specs=[
            pl.BlockSpec((BLK, D), lambda i: (i, 0)),
            pl.BlockSpec((BLK, D), lambda i: (i, 0)),
        ],
        out_shape=[
            jax.ShapeDtypeStruct((N, D), jnp.float32),
            jax.ShapeDtypeStruct((N, D), jnp.float32),
        ],
    )(X, agg2, cnt2r, Wx_l, Wx_r, bx, bh, w_c, b_gate)
    return H, C


# speedup vs baseline: 14.2495x; 1.8300x over previous
import functools

import jax
import jax.numpy as jnp
from jax import lax
from jax.experimental import pallas as pl
from jax.experimental.pallas import tpu as pltpu
from jax.experimental.pallas import tpu_sc as plsc

N = 10000
E = 320000
D = 128
NP = 10240
NC = 2
NS = 16
NW = NC * NS
EPW = E // NW
K = 80
NCHUNK = EPW // K
ROWS_PER_TILE = NP // NS


def _sc_body(x_hbm, src_hbm, dst_hbm, agg_out, cnt_out,
             agg_sh, cnt_sh, sidx0, didx0, sidx1, didx1, rows0, rows1,
             ones_v, zcnt, gsem0, gsem1, isem0, isem1):
    cid = lax.axis_index("c")
    sid = lax.axis_index("s")
    wid = sid * NC + cid

    z16 = jnp.zeros((16,), jnp.float32)

    def _zr(i, _):
        for j in range(8):
            rows0[i, pl.ds(j * 16, 16)] = z16
        return 0
    lax.fori_loop(0, K, _zr, 0)

    def _zc(i, _):
        zcnt[pl.ds(i * 16, 16)] = z16
        return 0
    lax.fori_loop(0, ROWS_PER_TILE // 16, _zc, 0)

    for j in range(K // 16):
        ones_v[pl.ds(j * 16, 16)] = jnp.ones((16,), jnp.float32)

    row0 = sid * ROWS_PER_TILE
    for t in range(ROWS_PER_TILE // K):
        pltpu.sync_copy(rows0, agg_sh.at[pl.ds(row0 + t * K, K)])
    pltpu.sync_copy(zcnt, cnt_sh.at[pl.ds(row0, ROWS_PER_TILE)])
    plsc.subcore_barrier()

    ebase = wid * EPW

    def _idx_start(j, sb, db, sem):
        pltpu.async_copy(src_hbm.at[pl.ds(ebase + j * K, K)], sb, sem)
        pltpu.async_copy(dst_hbm.at[pl.ds(ebase + j * K, K)], db, sem)

    def _idx_wait(j, sb, db, sem):
        pltpu.make_async_copy(src_hbm.at[pl.ds(ebase + j * K, K)], sb, sem).wait()
        pltpu.make_async_copy(dst_hbm.at[pl.ds(ebase + j * K, K)], db, sem).wait()

    def _gather_start(sb, rows, sem):
        pltpu.async_copy(x_hbm.at[sb], rows, sem)

    def _gather_wait(sb, rows, sem):
        pltpu.make_async_copy(x_hbm.at[sb], rows, sem).wait()

    def _scatter(rows, didx):
        pltpu.sync_copy(rows, agg_sh.at[didx], add=True)
        pltpu.sync_copy(ones_v, cnt_sh.at[didx], add=True)

    _idx_start(0, sidx0, didx0, isem0)
    _idx_wait(0, sidx0, didx0, isem0)
    _gather_start(sidx0, rows0, gsem0)
    _idx_start(1, sidx1, didx1, isem1)

    def _pair(i, _):
        j = 2 * i
        _idx_wait(j + 1, sidx1, didx1, isem1)
        _gather_start(sidx1, rows1, gsem1)
        _gather_wait(sidx0, rows0, gsem0)
        _scatter(rows0, didx0)
        _idx_start(j + 2, sidx0, didx0, isem0)
        _idx_wait(j + 2, sidx0, didx0, isem0)
        _gather_start(sidx0, rows0, gsem0)
        _gather_wait(sidx1, rows1, gsem1)
        _scatter(rows1, didx1)
        _idx_start(j + 3, sidx1, didx1, isem1)
        return 0
    lax.fori_loop(0, (NCHUNK - 3) // 2, _pair, 0)

    _idx_wait(NCHUNK - 2, sidx1, didx1, isem1)
    _gather_start(sidx1, rows1, gsem1)
    _gather_wait(sidx0, rows0, gsem0)
    _scatter(rows0, didx0)
    _idx_start(NCHUNK - 1, sidx0, didx0, isem0)
    _idx_wait(NCHUNK - 1, sidx0, didx0, isem0)
    _gather_start(sidx0, rows0, gsem0)
    _gather_wait(sidx1, rows1, gsem1)
    _scatter(rows1, didx1)
    _gather_wait(sidx0, rows0, gsem0)
    _scatter(rows0, didx0)

    plsc.subcore_barrier()

    pltpu.sync_copy(agg_sh.at[pl.ds(row0, ROWS_PER_TILE)],
                    agg_out.at[cid, pl.ds(row0, ROWS_PER_TILE)])
    pltpu.sync_copy(cnt_sh.at[pl.ds(row0, ROWS_PER_TILE)],
                    cnt_out.at[cid, pl.ds(row0, ROWS_PER_TILE)])


@functools.cache
def _sc_segment_sum():
    return pl.kernel(
        _sc_body,
        out_type=(jax.ShapeDtypeStruct((NC, NP, D), jnp.float32),
                  jax.ShapeDtypeStruct((NC, NP), jnp.float32)),
        mesh=plsc.VectorSubcoreMesh(core_axis_name="c", subcore_axis_name="s",
                                    num_cores=NC, num_subcores=NS),
        scratch_types=(
            pltpu.VMEM_SHARED((NP, D), jnp.float32),
            pltpu.VMEM_SHARED((NP,), jnp.float32),
            pltpu.VMEM((K,), jnp.int32),
            pltpu.VMEM((K,), jnp.int32),
            pltpu.VMEM((K,), jnp.int32),
            pltpu.VMEM((K,), jnp.int32),
            pltpu.VMEM((K, D), jnp.float32),
            pltpu.VMEM((K, D), jnp.float32),
            pltpu.VMEM((K,), jnp.float32),
            pltpu.VMEM((ROWS_PER_TILE,), jnp.float32),
            pltpu.SemaphoreType.DMA,
            pltpu.SemaphoreType.DMA,
            pltpu.SemaphoreType.DMA,
            pltpu.SemaphoreType.DMA,
        ),
    )


BLK = 2000


def _dense_body(x_ref, agg_ref, cnt_ref, wl_ref, wr_ref, bx_ref, bh_ref,
                wc_ref, bg_ref, h_ref, c_ref):
    X = x_ref[...]
    agg = agg_ref[0] + agg_ref[1]
    cnt = cnt_ref[0] + cnt_ref[1]
    mean = agg / jnp.maximum(cnt, 1.0)

    def gate(k):
        U = (jnp.dot(mean, wl_ref[k], preferred_element_type=jnp.float32)
             + jnp.dot(X, wr_ref[k], preferred_element_type=jnp.float32)
             + bx_ref[k])
        nrm = jnp.sqrt(jnp.sum(U * U, axis=1, keepdims=True))
        S = U / jnp.maximum(nrm, 1e-12)
        bhk = bh_ref[k]
        bnrm = jnp.sqrt(jnp.sum(bhk * bhk))
        return S + bhk / jnp.maximum(bnrm, 1e-12) + bg_ref[k]

    I = jax.nn.sigmoid(gate(0))
    T = jnp.tanh(gate(2))
    C = I * T
    O = jax.nn.sigmoid(gate(3) + wc_ref[2] * C)
    h_ref[...] = O * jnp.tanh(C)
    c_ref[...] = C


def kernel(X, edge_index, Wx_l, Wx_r, bx, Wh_l, Wh_r, bh, w_c, b_gate):
    del Wh_l, Wh_r
    edges = edge_index.astype(jnp.int32)
    agg2, cnt2 = _sc_segment_sum()(X, edges[0], edges[1])
    cnt2r = cnt2.reshape(NC, NP, 1)

    H, C = pl.pallas_call(
        _dense_body,
        grid=(N // BLK,),
        in_specs=[
            pl.BlockSpec((BLK, D), lambda i: (i, 0)),
            pl.BlockSpec((NC, BLK, D), lambda i: (0, i, 0)),
            pl.BlockSpec((NC, BLK, 1), lambda i: (0, i, 0)),
            pl.BlockSpec((4, D, D), lambda i: (0, 0, 0)),
            pl.BlockSpec((4, D, D), lambda i: (0, 0, 0)),
            pl.BlockSpec((4, D), lambda i: (0, 0)),
            pl.BlockSpec((4, D), lambda i: (0, 0)),
            pl.BlockSpec((3, D), lambda i: (0, 0)),
            pl.BlockSpec((4, D), lambda i: (0, 0)),
        ],
        out_specs=[
            pl.BlockSpec((BLK, D), lambda i: (i, 0)),
            pl.BlockSpec((BLK, D), lambda i: (i, 0)),
        ],
        out_shape=[
            jax.ShapeDtypeStruct((N, D), jnp.float32),
            jax.ShapeDtypeStruct((N, D), jnp.float32),
        ],
    )(X, agg2, cnt2r, Wx_l, Wx_r, bx, bh, w_c, b_gate)
    return H, C


# speedup vs baseline: 16.9085x; 1.1866x over previous
import functools

import jax
import jax.numpy as jnp
from jax import lax
from jax.experimental import pallas as pl
from jax.experimental.pallas import tpu as pltpu
from jax.experimental.pallas import tpu_sc as plsc

N = 10000
E = 320000
D = 128
NP = 10240
NC = 2
NS = 16
NW = NC * NS
K = 128
NCHUNK = 79
NREAL = 78
NPAD = NW * K - (E - NREAL * K * NW)
ROWS_PER_TILE = NP // NS


def _sc_body(x_hbm, edges_hbm, epad_hbm, agg_out, cnt_out,
             agg_sh, cnt_sh, ebuf0, ebuf1, dscr0, dscr1, rows0, rows1,
             ones_v, zcnt,
             gsem0, gsem1, ssem0, ssem1, csem0, csem1, isem0, isem1):
    cid = lax.axis_index("c")
    sid = lax.axis_index("s")
    wid = sid * NC + cid

    z16 = jnp.zeros((16,), jnp.float32)

    def _zr(i, _):
        for j in range(8):
            rows0[i, pl.ds(j * 16, 16)] = z16
        return 0
    lax.fori_loop(0, K, _zr, 0)

    def _zc(i, _):
        zcnt[pl.ds(i * 16, 16)] = z16
        return 0
    lax.fori_loop(0, ROWS_PER_TILE // 16, _zc, 0)

    for j in range(K // 16):
        ones_v[pl.ds(j * 16, 16)] = jnp.ones((16,), jnp.float32)

    row0 = sid * ROWS_PER_TILE
    for t in range(ROWS_PER_TILE // K):
        pltpu.sync_copy(rows0, agg_sh.at[pl.ds(row0 + t * K, K)])
    pltpu.sync_copy(zcnt, cnt_sh.at[pl.ds(row0, ROWS_PER_TILE)])
    plsc.subcore_barrier()

    def _idx_ref(j):
        if isinstance(j, int) and j >= NREAL:
            return epad_hbm.at[:, pl.ds(pl.multiple_of(wid * K, K), K)]
        off = pl.multiple_of((wid * NREAL + j) * K, K)
        return edges_hbm.at[:, pl.ds(off, K)]

    def _idx_start(j, ebuf, sem):
        pltpu.async_copy(_idx_ref(j), ebuf, sem)

    def _idx_wait(j, ebuf, sem):
        pltpu.make_async_copy(_idx_ref(j), ebuf, sem).wait()

    def _gather_start(ebuf, rows, sem):
        pltpu.async_copy(x_hbm.at[ebuf.at[0]], rows, sem)

    def _gather_wait(ebuf, rows, sem):
        pltpu.make_async_copy(x_hbm.at[ebuf.at[0]], rows, sem).wait()

    def _scatter_start(rows, dscr, ebuf, ssem, csem):
        for t in range(K // 16):
            dscr[pl.ds(t * 16, 16)] = ebuf[1, pl.ds(t * 16, 16)]
        pltpu.async_copy(rows, agg_sh.at[dscr], ssem, add=True)
        pltpu.async_copy(ones_v, cnt_sh.at[dscr], csem, add=True)

    def _scatter_wait(rows, dscr, ssem, csem):
        pltpu.make_async_copy(rows, agg_sh.at[dscr], ssem).wait()
        pltpu.make_async_copy(ones_v, cnt_sh.at[dscr], csem).wait()

    _idx_start(0, ebuf0, isem0)
    _idx_wait(0, ebuf0, isem0)
    _gather_start(ebuf0, rows0, gsem0)
    _idx_start(1, ebuf1, isem1)

    _gather_wait(ebuf0, rows0, gsem0)
    _scatter_start(rows0, dscr0, ebuf0, ssem0, csem0)
    _idx_start(2, ebuf0, isem0)
    _idx_wait(1, ebuf1, isem1)
    _gather_start(ebuf1, rows1, gsem1)

    def _half(j, ebuf_c, dscr_c, rows_c, gsem_c, ssem_c, csem_c, isem_c,
              ebuf_o, dscr_o, rows_o, gsem_o, ssem_o, csem_o, isem_o,
              start_idx=True, start_gather=True):
        _gather_wait(ebuf_c, rows_c, gsem_c)
        _scatter_start(rows_c, dscr_c, ebuf_c, ssem_c, csem_c)
        if start_idx:
            _idx_start(j + 2, ebuf_c, isem_c)
        _scatter_wait(rows_o, dscr_o, ssem_o, csem_o)
        if start_gather:
            _idx_wait(j + 1, ebuf_o, isem_o)
            _gather_start(ebuf_o, rows_o, gsem_o)

    s0 = (ebuf0, dscr0, rows0, gsem0, ssem0, csem0, isem0)
    s1 = (ebuf1, dscr1, rows1, gsem1, ssem1, csem1, isem1)

    def _pair(i, _):
        j = 1 + 2 * i
        _half(j, *s1, *s0)
        _half(j + 1, *s0, *s1)
        return 0
    lax.fori_loop(0, 37, _pair, 0)

    _half(75, *s1, *s0)
    _half(76, *s0, *s1)
    _half(77, *s1, *s0, start_idx=False)
    _half(78, *s0, *s1, start_idx=False, start_gather=False)
    _scatter_wait(rows0, dscr0, ssem0, csem0)

    plsc.subcore_barrier()

    pltpu.sync_copy(agg_sh.at[pl.ds(row0, ROWS_PER_TILE)],
                    agg_out.at[cid, pl.ds(row0, ROWS_PER_TILE)])
    pltpu.sync_copy(cnt_sh.at[pl.ds(row0, ROWS_PER_TILE)],
                    cnt_out.at[cid, pl.ds(row0, ROWS_PER_TILE)])


@functools.cache
def _sc_segment_sum():
    return pl.kernel(
        _sc_body,
        out_type=(jax.ShapeDtypeStruct((NC, NP, D), jnp.float32),
                  jax.ShapeDtypeStruct((NC, NP), jnp.float32)),
        mesh=plsc.VectorSubcoreMesh(core_axis_name="c", subcore_axis_name="s",
                                    num_cores=NC, num_subcores=NS),
        scratch_types=(
            pltpu.VMEM_SHARED((NP, D), jnp.float32),
            pltpu.VMEM_SHARED((NP,), jnp.float32),
            pltpu.VMEM((2, K), jnp.int32),
            pltpu.VMEM((2, K), jnp.int32),
            pltpu.VMEM((K,), jnp.int32),
            pltpu.VMEM((K,), jnp.int32),
            pltpu.VMEM((K, D), jnp.float32),
            pltpu.VMEM((K, D), jnp.float32),
            pltpu.VMEM((K,), jnp.float32),
            pltpu.VMEM((ROWS_PER_TILE,), jnp.float32),
            pltpu.SemaphoreType.DMA,
            pltpu.SemaphoreType.DMA,
            pltpu.SemaphoreType.DMA,
            pltpu.SemaphoreType.DMA,
            pltpu.SemaphoreType.DMA,
            pltpu.SemaphoreType.DMA,
            pltpu.SemaphoreType.DMA,
            pltpu.SemaphoreType.DMA,
        ),
    )


BLK = 2000


def _dense_body(x_ref, agg_ref, cnt_ref, wl_ref, wr_ref, bx_ref, bh_ref,
                wc_ref, bg_ref, h_ref, c_ref):
    X = x_ref[...]
    agg = agg_ref[0] + agg_ref[1]
    cnt = cnt_ref[0] + cnt_ref[1]
    mean = agg / jnp.maximum(cnt, 1.0)

    def gate(k):
        U = (jnp.dot(mean, wl_ref[k], preferred_element_type=jnp.float32)
             + jnp.dot(X, wr_ref[k], preferred_element_type=jnp.float32)
             + bx_ref[k])
        nrm = jnp.sqrt(jnp.sum(U * U, axis=1, keepdims=True))
        S = U / jnp.maximum(nrm, 1e-12)
        bhk = bh_ref[k]
        bnrm = jnp.sqrt(jnp.sum(bhk * bhk))
        return S + bhk / jnp.maximum(bnrm, 1e-12) + bg_ref[k]

    I = jax.nn.sigmoid(gate(0))
    T = jnp.tanh(gate(2))
    C = I * T
    O = jax.nn.sigmoid(gate(3) + wc_ref[2] * C)
    h_ref[...] = O * jnp.tanh(C)
    c_ref[...] = C


def kernel(X, edge_index, Wx_l, Wx_r, bx, Wh_l, Wh_r, bh, w_c, b_gate):
    del Wh_l, Wh_r
    edges = edge_index.astype(jnp.int32)
    ar = jnp.arange(NPAD, dtype=jnp.int32)
    fake = jnp.stack([(ar * 997) % N, N + ar % (NP - N)])
    epad = jnp.concatenate([edges[:, NREAL * K * NW:], fake], axis=1)
    agg2, cnt2 = _sc_segment_sum()(X, edges, epad)
    cnt2r = cnt2.reshape(NC, NP, 1)

    H, C = pl.pallas_call(
        _dense_body,
        grid=(N // BLK,),
        in_specs=[
            pl.BlockSpec((BLK, D), lambda i: (i, 0)),
            pl.BlockSpec((NC, BLK, D), lambda i: (0, i, 0)),
            pl.BlockSpec((NC, BLK, 1), lambda i: (0, i, 0)),
            pl.BlockSpec((4, D, D), lambda i: (0, 0, 0)),
            pl.BlockSpec((4, D, D), lambda i: (0, 0, 0)),
            pl.BlockSpec((4, D), lambda i: (0, 0)),
            pl.BlockSpec((4, D), lambda i: (0, 0)),
            pl.BlockSpec((3, D), lambda i: (0, 0)),
            pl.BlockSpec((4, D), lambda i: (0, 0)),
        ],
        out_specs=[
            pl.BlockSpec((BLK, D), lambda i: (i, 0)),
            pl.BlockSpec((BLK, D), lambda i: (i, 0)),
        ],
        out_shape=[
            jax.ShapeDtypeStruct((N, D), jnp.float32),
            jax.ShapeDtypeStruct((N, D), jnp.float32),
        ],
    )(X, agg2, cnt2r, Wx_l, Wx_r, bx, bh, w_c, b_gate)
    return H, C


# speedup vs baseline: 17.0470x; 1.0082x over previous
import functools

import jax
import jax.numpy as jnp
from jax import lax
from jax.experimental import pallas as pl
from jax.experimental.pallas import tpu as pltpu
from jax.experimental.pallas import tpu_sc as plsc

N = 10000
E = 320000
D = 128
NP = 10240
NC = 2
NS = 16
NW = NC * NS
K = 128
NREAL = 78
TAIL0 = NREAL * K * NW
TAILN = E - TAIL0
ROWS_PER_TILE = NP // NS


def _sc_body(x_hbm, edges_hbm, agg_out, cnt_out,
             agg_sh, cnt_sh, ebuf0, ebuf1, dscr0, dscr1, rows0, rows1,
             ebuf_t, sidx_t, didx_t, rows_t, ones_v, zcnt,
             gsem0, gsem1, ssem0, ssem1, csem0, csem1, isem0, isem1):
    cid = lax.axis_index("c")
    sid = lax.axis_index("s")
    wid = sid * NC + cid

    z16 = jnp.zeros((16,), jnp.float32)

    def _zr(i, _):
        for j in range(8):
            rows0[i, pl.ds(j * 16, 16)] = z16
        return 0
    lax.fori_loop(0, K, _zr, 0)

    def _zc(i, _):
        zcnt[pl.ds(i * 16, 16)] = z16
        return 0
    lax.fori_loop(0, ROWS_PER_TILE // 16, _zc, 0)

    for j in range(K // 16):
        ones_v[pl.ds(j * 16, 16)] = jnp.ones((16,), jnp.float32)

    row0 = sid * ROWS_PER_TILE
    for t in range(ROWS_PER_TILE // K):
        pltpu.sync_copy(rows0, agg_sh.at[pl.ds(row0 + t * K, K)])
    pltpu.sync_copy(zcnt, cnt_sh.at[pl.ds(row0, ROWS_PER_TILE)])
    plsc.subcore_barrier()

    def _idx_ref(j):
        off = pl.multiple_of((wid * NREAL + j) * K, K)
        return edges_hbm.at[:, pl.ds(off, K)]

    def _idx_start(j, ebuf, sem):
        pltpu.async_copy(_idx_ref(j), ebuf, sem)

    def _idx_wait(j, ebuf, sem):
        pltpu.make_async_copy(_idx_ref(j), ebuf, sem).wait()

    def _gather_start(ebuf, rows, sem):
        pltpu.async_copy(x_hbm.at[ebuf.at[0]], rows, sem)

    def _gather_wait(ebuf, rows, sem):
        pltpu.make_async_copy(x_hbm.at[ebuf.at[0]], rows, sem).wait()

    def _scatter_start(rows, dscr, ebuf, ssem, csem):
        for t in range(K // 16):
            dscr[pl.ds(t * 16, 16)] = ebuf[1, pl.ds(t * 16, 16)]
        pltpu.async_copy(rows, agg_sh.at[dscr], ssem, add=True)
        pltpu.async_copy(ones_v, cnt_sh.at[dscr], csem, add=True)

    def _scatter_wait(rows, dscr, ssem, csem):
        pltpu.make_async_copy(rows, agg_sh.at[dscr], ssem).wait()
        pltpu.make_async_copy(ones_v, cnt_sh.at[dscr], csem).wait()

    _idx_start(0, ebuf0, isem0)
    _idx_wait(0, ebuf0, isem0)
    _gather_start(ebuf0, rows0, gsem0)
    _idx_start(1, ebuf1, isem1)

    _gather_wait(ebuf0, rows0, gsem0)
    _scatter_start(rows0, dscr0, ebuf0, ssem0, csem0)
    _idx_start(2, ebuf0, isem0)
    _idx_wait(1, ebuf1, isem1)
    _gather_start(ebuf1, rows1, gsem1)

    def _half(j, ebuf_c, dscr_c, rows_c, gsem_c, ssem_c, csem_c, isem_c,
              ebuf_o, dscr_o, rows_o, gsem_o, ssem_o, csem_o, isem_o,
              start_idx=True, start_gather=True):
        _gather_wait(ebuf_c, rows_c, gsem_c)
        _scatter_start(rows_c, dscr_c, ebuf_c, ssem_c, csem_c)
        if start_idx:
            _idx_start(j + 2, ebuf_c, isem_c)
        _scatter_wait(rows_o, dscr_o, ssem_o, csem_o)
        if start_gather:
            _idx_wait(j + 1, ebuf_o, isem_o)
            _gather_start(ebuf_o, rows_o, gsem_o)

    s0 = (ebuf0, dscr0, rows0, gsem0, ssem0, csem0, isem0)
    s1 = (ebuf1, dscr1, rows1, gsem1, ssem1, csem1, isem1)

    def _pair(i, _):
        j = 1 + 2 * i
        _half(j, *s1, *s0)
        _half(j + 1, *s0, *s1)
        return 0
    lax.fori_loop(0, 37, _pair, 0)

    _half(75, *s1, *s0)
    _half(76, *s0, *s1, start_idx=False)
    _half(77, *s1, *s0, start_idx=False, start_gather=False)
    _scatter_wait(rows1, dscr1, ssem1, csem1)

    pltpu.sync_copy(edges_hbm.at[:, pl.ds(TAIL0, TAILN)], ebuf_t)
    sidx_t[...] = ebuf_t[0, pl.ds(wid * 16, 16)]
    didx_t[...] = ebuf_t[1, pl.ds(wid * 16, 16)]
    pltpu.sync_copy(x_hbm.at[sidx_t], rows_t)
    pltpu.sync_copy(rows_t, agg_sh.at[didx_t], add=True)
    pltpu.sync_copy(ones_v.at[pl.ds(0, 16)], cnt_sh.at[didx_t], add=True)

    plsc.subcore_barrier()

    pltpu.sync_copy(agg_sh.at[pl.ds(row0, ROWS_PER_TILE)],
                    agg_out.at[cid, pl.ds(row0, ROWS_PER_TILE)])
    pltpu.sync_copy(cnt_sh.at[pl.ds(row0, ROWS_PER_TILE)],
                    cnt_out.at[cid, pl.ds(row0, ROWS_PER_TILE)])


@functools.cache
def _sc_segment_sum():
    return pl.kernel(
        _sc_body,
        out_type=(jax.ShapeDtypeStruct((NC, NP, D), jnp.float32),
                  jax.ShapeDtypeStruct((NC, NP), jnp.float32)),
        mesh=plsc.VectorSubcoreMesh(core_axis_name="c", subcore_axis_name="s",
                                    num_cores=NC, num_subcores=NS),
        scratch_types=(
            pltpu.VMEM_SHARED((NP, D), jnp.float32),
            pltpu.VMEM_SHARED((NP,), jnp.float32),
            pltpu.VMEM((2, K), jnp.int32),
            pltpu.VMEM((2, K), jnp.int32),
            pltpu.VMEM((K,), jnp.int32),
            pltpu.VMEM((K,), jnp.int32),
            pltpu.VMEM((K, D), jnp.float32),
            pltpu.VMEM((K, D), jnp.float32),
            pltpu.VMEM((2, TAILN), jnp.int32),
            pltpu.VMEM((16,), jnp.int32),
            pltpu.VMEM((16,), jnp.int32),
            pltpu.VMEM((16, D), jnp.float32),
            pltpu.VMEM((K,), jnp.float32),
            pltpu.VMEM((ROWS_PER_TILE,), jnp.float32),
            pltpu.SemaphoreType.DMA,
            pltpu.SemaphoreType.DMA,
            pltpu.SemaphoreType.DMA,
            pltpu.SemaphoreType.DMA,
            pltpu.SemaphoreType.DMA,
            pltpu.SemaphoreType.DMA,
            pltpu.SemaphoreType.DMA,
            pltpu.SemaphoreType.DMA,
        ),
    )


BLK = 2000


def _dense_body(x_ref, agg_ref, cnt_ref, wl_ref, wr_ref, bx_ref, bh_ref,
                wc_ref, bg_ref, h_ref, c_ref):
    X = x_ref[...]
    agg = agg_ref[0] + agg_ref[1]
    cnt = cnt_ref[0] + cnt_ref[1]
    mean = agg / jnp.maximum(cnt, 1.0)

    def gate(k):
        U = (jnp.dot(mean, wl_ref[k], preferred_element_type=jnp.float32)
             + jnp.dot(X, wr_ref[k], preferred_element_type=jnp.float32)
             + bx_ref[k])
        nrm = jnp.sqrt(jnp.sum(U * U, axis=1, keepdims=True))
        S = U / jnp.maximum(nrm, 1e-12)
        bhk = bh_ref[k]
        bnrm = jnp.sqrt(jnp.sum(bhk * bhk))
        return S + bhk / jnp.maximum(bnrm, 1e-12) + bg_ref[k]

    I = jax.nn.sigmoid(gate(0))
    T = jnp.tanh(gate(2))
    C = I * T
    O = jax.nn.sigmoid(gate(3) + wc_ref[2] * C)
    h_ref[...] = O * jnp.tanh(C)
    c_ref[...] = C


def kernel(X, edge_index, Wx_l, Wx_r, bx, Wh_l, Wh_r, bh, w_c, b_gate):
    del Wh_l, Wh_r
    edges = edge_index.astype(jnp.int32)
    agg2, cnt2 = _sc_segment_sum()(X, edges)
    cnt2r = cnt2.reshape(NC, NP, 1)

    H, C = pl.pallas_call(
        _dense_body,
        grid=(N // BLK,),
        in_specs=[
            pl.BlockSpec((BLK, D), lambda i: (i, 0)),
            pl.BlockSpec((NC, BLK, D), lambda i: (0, i, 0)),
            pl.BlockSpec((NC, BLK, 1), lambda i: (0, i, 0)),
            pl.BlockSpec((4, D, D), lambda i: (0, 0, 0)),
            pl.BlockSpec((4, D, D), lambda i: (0, 0, 0)),
            pl.BlockSpec((4, D), lambda i: (0, 0)),
            pl.BlockSpec((4, D), lambda i: (0, 0)),
            pl.BlockSpec((3, D), lambda i: (0, 0)),
            pl.BlockSpec((4, D), lambda i: (0, 0)),
        ],
        out_specs=[
            pl.BlockSpec((BLK, D), lambda i: (i, 0)),
            pl.BlockSpec((BLK, D), lambda i: (i, 0)),
        ],
        out_shape=[
            jax.ShapeDtypeStruct((N, D), jnp.float32),
            jax.ShapeDtypeStruct((N, D), jnp.float32),
        ],
    )(X, agg2, cnt2r, Wx_l, Wx_r, bx, bh, w_c, b_gate)
    return H, C


# speedup vs baseline: 17.7590x; 1.0418x over previous
import functools

import jax
import jax.numpy as jnp
from jax import lax
from jax.experimental import pallas as pl
from jax.experimental.pallas import tpu as pltpu
from jax.experimental.pallas import tpu_sc as plsc

N = 10000
E = 320000
D = 128
NP = 10240
NC = 2
NS = 16
NW = NC * NS
K = 128
NREAL = 78
TAIL0 = NREAL * K * NW
TAILN = E - TAIL0
ROWS_PER_TILE = NP // NS


def _sc_body(x_hbm, edges_hbm, agg_out, cnt_out,
             agg_sh, cnt_sh, ebuf0, ebuf1, dscr0, dscr1, rows0, rows1,
             ebuf_t, sidx_t, didx_t, rows_t, ones_v, zcnt,
             gsem0, gsem1, ssem0, ssem1, csem0, csem1, isem0, isem1):
    cid = lax.axis_index("c")
    sid = lax.axis_index("s")
    wid = sid * NC + cid

    z16 = jnp.zeros((16,), jnp.float32)

    def _zr(i, _):
        for j in range(8):
            rows0[i, pl.ds(j * 16, 16)] = z16
        return 0
    lax.fori_loop(0, K, _zr, 0)

    def _zc(i, _):
        zcnt[pl.ds(i * 16, 16)] = z16
        return 0
    lax.fori_loop(0, ROWS_PER_TILE // 16, _zc, 0)

    for j in range(K // 16):
        ones_v[pl.ds(j * 16, 16)] = jnp.ones((16,), jnp.float32)

    row0 = sid * ROWS_PER_TILE
    for t in range(ROWS_PER_TILE // K):
        pltpu.sync_copy(rows0, agg_sh.at[pl.ds(row0 + t * K, K)])
    pltpu.sync_copy(zcnt, cnt_sh.at[pl.ds(row0, ROWS_PER_TILE)])
    plsc.subcore_barrier()

    def _idx_ref(j):
        off = pl.multiple_of((wid * NREAL + j) * K, K)
        return edges_hbm.at[:, pl.ds(off, K)]

    def _idx_start(j, ebuf, sem):
        pltpu.async_copy(_idx_ref(j), ebuf, sem)

    def _idx_wait(j, ebuf, sem):
        pltpu.make_async_copy(_idx_ref(j), ebuf, sem).wait()

    def _gather_start(ebuf, rows, sem):
        pltpu.async_copy(x_hbm.at[ebuf.at[0]], rows, sem)

    def _gather_wait(ebuf, rows, sem):
        pltpu.make_async_copy(x_hbm.at[ebuf.at[0]], rows, sem).wait()

    def _scatter_start(rows, dscr, ebuf, ssem, csem):
        for t in range(K // 16):
            dscr[pl.ds(t * 16, 16)] = ebuf[1, pl.ds(t * 16, 16)]
        pltpu.async_copy(rows, agg_sh.at[dscr], ssem, add=True)
        pltpu.async_copy(ones_v, cnt_sh.at[dscr], csem, add=True)

    def _scatter_wait(rows, dscr, ssem, csem):
        pltpu.make_async_copy(rows, agg_sh.at[dscr], ssem).wait()
        pltpu.make_async_copy(ones_v, cnt_sh.at[dscr], csem).wait()

    _idx_start(0, ebuf0, isem0)
    _idx_wait(0, ebuf0, isem0)
    _gather_start(ebuf0, rows0, gsem0)
    _idx_start(1, ebuf1, isem1)

    _gather_wait(ebuf0, rows0, gsem0)
    _scatter_start(rows0, dscr0, ebuf0, ssem0, csem0)
    _idx_start(2, ebuf0, isem0)
    _idx_wait(1, ebuf1, isem1)
    _gather_start(ebuf1, rows1, gsem1)

    def _half(j, ebuf_c, dscr_c, rows_c, gsem_c, ssem_c, csem_c, isem_c,
              ebuf_o, dscr_o, rows_o, gsem_o, ssem_o, csem_o, isem_o,
              start_idx=True, start_gather=True):
        _gather_wait(ebuf_c, rows_c, gsem_c)
        _scatter_start(rows_c, dscr_c, ebuf_c, ssem_c, csem_c)
        if start_idx:
            _idx_start(j + 2, ebuf_c, isem_c)
        _scatter_wait(rows_o, dscr_o, ssem_o, csem_o)
        if start_gather:
            _idx_wait(j + 1, ebuf_o, isem_o)
            _gather_start(ebuf_o, rows_o, gsem_o)

    s0 = (ebuf0, dscr0, rows0, gsem0, ssem0, csem0, isem0)
    s1 = (ebuf1, dscr1, rows1, gsem1, ssem1, csem1, isem1)

    def _pair(i, _):
        j = 1 + 2 * i
        _half(j, *s1, *s0)
        _half(j + 1, *s0, *s1)
        return 0
    lax.fori_loop(0, 37, _pair, 0)

    _half(75, *s1, *s0)
    _half(76, *s0, *s1, start_idx=False)
    _half(77, *s1, *s0, start_idx=False, start_gather=False)
    _scatter_wait(rows1, dscr1, ssem1, csem1)

    pltpu.sync_copy(edges_hbm.at[:, pl.ds(TAIL0, TAILN)], ebuf_t)
    sidx_t[...] = ebuf_t[0, pl.ds(wid * 16, 16)]
    didx_t[...] = ebuf_t[1, pl.ds(wid * 16, 16)]
    pltpu.sync_copy(x_hbm.at[sidx_t], rows_t)
    pltpu.sync_copy(rows_t, agg_sh.at[didx_t], add=True)
    pltpu.sync_copy(ones_v.at[pl.ds(0, 16)], cnt_sh.at[didx_t], add=True)

    plsc.subcore_barrier()

    pltpu.sync_copy(agg_sh.at[pl.ds(row0, ROWS_PER_TILE)],
                    agg_out.at[cid, pl.ds(row0, ROWS_PER_TILE)])
    pltpu.sync_copy(cnt_sh.at[pl.ds(row0, ROWS_PER_TILE)],
                    cnt_out.at[cid, pl.ds(row0, ROWS_PER_TILE)])


@functools.cache
def _sc_segment_sum():
    return pl.kernel(
        _sc_body,
        out_type=(jax.ShapeDtypeStruct((NC, NP, D), jnp.float32),
                  jax.ShapeDtypeStruct((NC, NP), jnp.float32)),
        mesh=plsc.VectorSubcoreMesh(core_axis_name="c", subcore_axis_name="s",
                                    num_cores=NC, num_subcores=NS),
        scratch_types=(
            pltpu.VMEM_SHARED((NP, D), jnp.float32),
            pltpu.VMEM_SHARED((NP,), jnp.float32),
            pltpu.VMEM((2, K), jnp.int32),
            pltpu.VMEM((2, K), jnp.int32),
            pltpu.VMEM((K,), jnp.int32),
            pltpu.VMEM((K,), jnp.int32),
            pltpu.VMEM((K, D), jnp.float32),
            pltpu.VMEM((K, D), jnp.float32),
            pltpu.VMEM((2, TAILN), jnp.int32),
            pltpu.VMEM((16,), jnp.int32),
            pltpu.VMEM((16,), jnp.int32),
            pltpu.VMEM((16, D), jnp.float32),
            pltpu.VMEM((K,), jnp.float32),
            pltpu.VMEM((ROWS_PER_TILE,), jnp.float32),
            pltpu.SemaphoreType.DMA,
            pltpu.SemaphoreType.DMA,
            pltpu.SemaphoreType.DMA,
            pltpu.SemaphoreType.DMA,
            pltpu.SemaphoreType.DMA,
            pltpu.SemaphoreType.DMA,
            pltpu.SemaphoreType.DMA,
            pltpu.SemaphoreType.DMA,
        ),
    )


BLK = 2560


def _dense_body(x_ref, agg_ref, cnt_ref, wl_ref, wr_ref, bx_ref, bh_ref,
                wc_ref, bg_ref, h_ref, c_ref):
    X = x_ref[...]
    agg = agg_ref[0] + agg_ref[1]
    cnt = (cnt_ref[0] + cnt_ref[1])[:, None]
    mean = agg / jnp.maximum(cnt, 1.0)

    def gate(k):
        U = (jnp.dot(mean, wl_ref[k], preferred_element_type=jnp.float32)
             + jnp.dot(X, wr_ref[k], preferred_element_type=jnp.float32)
             + bx_ref[k])
        nrm = jnp.sqrt(jnp.sum(U * U, axis=1, keepdims=True))
        S = U / jnp.maximum(nrm, 1e-12)
        bhk = bh_ref[k]
        bnrm = jnp.sqrt(jnp.sum(bhk * bhk))
        return S + bhk / jnp.maximum(bnrm, 1e-12) + bg_ref[k]

    I = jax.nn.sigmoid(gate(0))
    T = jnp.tanh(gate(2))
    C = I * T
    O = jax.nn.sigmoid(gate(3) + wc_ref[2] * C)
    h_ref[...] = O * jnp.tanh(C)
    c_ref[...] = C


def kernel(X, edge_index, Wx_l, Wx_r, bx, Wh_l, Wh_r, bh, w_c, b_gate):
    del Wh_l, Wh_r
    edges = edge_index.astype(jnp.int32)
    agg2, cnt2 = _sc_segment_sum()(X, edges)

    H, C = pl.pallas_call(
        _dense_body,
        grid=(NP // BLK,),
        in_specs=[
            pl.BlockSpec((BLK, D), lambda i: (i, 0)),
            pl.BlockSpec((NC, BLK, D), lambda i: (0, i, 0)),
            pl.BlockSpec((NC, BLK), lambda i: (0, i)),
            pl.BlockSpec((4, D, D), lambda i: (0, 0, 0)),
            pl.BlockSpec((4, D, D), lambda i: (0, 0, 0)),
            pl.BlockSpec((4, D), lambda i: (0, 0)),
            pl.BlockSpec((4, D), lambda i: (0, 0)),
            pl.BlockSpec((3, D), lambda i: (0, 0)),
            pl.BlockSpec((4, D), lambda i: (0, 0)),
        ],
        out_specs=[
            pl.BlockSpec((BLK, D), lambda i: (i, 0)),
            pl.BlockSpec((BLK, D), lambda i: (i, 0)),
        ],
        out_shape=[
            jax.ShapeDtypeStruct((N, D), jnp.float32),
            jax.ShapeDtypeStruct((N, D), jnp.float32),
        ],
    )(X, agg2, cnt2, Wx_l, Wx_r, bx, bh, w_c, b_gate)
    return H, C
